# baseline (speedup 1.0000x reference)
import functools

import jax
import jax.numpy as jnp
from jax import lax
from jax.experimental import pallas as pl
from jax.experimental.pallas import tpu as pltpu


def _shift_lanes(a, s):
    if s > 0:
        return jnp.pad(a[:, s:], ((0, 0), (0, s)))
    if s < 0:
        return jnp.pad(a[:, :s], ((0, 0), (-s, 0)))
    return a


def _fused_conv_stats_kernel(x_ref, w1_ref, b1_ref, w3_ref, b3_ref,
                             x3_ref, st_ref, *, mid, height, width):
    hw = height * width
    xf = x_ref[0].reshape(x_ref.shape[1], hw)
    x0 = jnp.dot(w1_ref[...], xf,
                 preferred_element_type=jnp.float32) + b1_ref[...]
    x1 = x0[:mid]

    pos = lax.broadcasted_iota(jnp.int32, (1, hw), 1)
    hidx = pos // width
    widx = pos - hidx * width
    taps = []
    for dh in (-1, 0, 1):
        for dw in (-1, 0, 1):
            shifted = _shift_lanes(x1, dh * width + dw)
            ok_h = jnp.logical_and(hidx + dh >= 0, hidx + dh < height)
            ok_w = jnp.logical_and(widx + dw >= 0, widx + dw < width)
            mask = jnp.logical_and(ok_h, ok_w)
            taps.append(jnp.where(mask, shifted, 0.0))
    patch = jnp.concatenate(taps, axis=0)
    y = jnp.dot(w3_ref[...], patch,
                preferred_element_type=jnp.float32) + b3_ref[...]

    x3 = jnp.concatenate([y, x0[mid:]], axis=0)
    x3_ref[0] = x3
    st_ref[0] = jnp.concatenate(
        [jnp.sum(x3, axis=1, keepdims=True),
         jnp.sum(x3 * x3, axis=1, keepdims=True)], axis=1)


def _bn_residual_kernel(x_ref, x3_ref, scale_ref, shift_ref, o_ref):
    bn, C, H, W = o_ref.shape
    x3 = x3_ref[...].reshape(bn, C, H, W)
    o_ref[...] = x_ref[...] + x3 * scale_ref[...] + shift_ref[...]


def kernel(x, w1_t, b1, w3_t, b3, gamma, beta, eps=1e-5):
    N, C, H, W = x.shape
    mid = C // 2
    HW = H * W
    M = N * HW

    xf = x.astype(jnp.float32)
    w1 = w1_t[:, :, 0, 0].astype(jnp.float32)
    b1c = b1.reshape(C, 1).astype(jnp.float32)
    w3 = jnp.transpose(w3_t, (0, 2, 3, 1)).reshape(mid, 9 * mid)
    w3 = w3.astype(jnp.float32)
    b3c = b3.reshape(mid, 1).astype(jnp.float32)

    kern = functools.partial(_fused_conv_stats_kernel,
                             mid=mid, height=H, width=W)
    x3, stats = pl.pallas_call(
        kern,
        out_shape=(jax.ShapeDtypeStruct((N, C, HW), jnp.float32),
                   jax.ShapeDtypeStruct((N, C, 2), jnp.float32)),
        grid=(N,),
        in_specs=[
            pl.BlockSpec((1, C, H, W), lambda n: (n, 0, 0, 0)),
            pl.BlockSpec((C, C), lambda n: (0, 0)),
            pl.BlockSpec((C, 1), lambda n: (0, 0)),
            pl.BlockSpec((mid, 9 * mid), lambda n: (0, 0)),
            pl.BlockSpec((mid, 1), lambda n: (0, 0)),
        ],
        out_specs=(
            pl.BlockSpec((1, C, HW), lambda n: (n, 0, 0)),
            pl.BlockSpec((1, C, 2), lambda n: (n, 0, 0)),
        ),
        compiler_params=pltpu.CompilerParams(
            dimension_semantics=("parallel",)),
    )(xf, w1, b1c, w3, b3c)

    tot = jnp.sum(stats, axis=0)
    mean = tot[:, 0] / M
    var = tot[:, 1] / M - mean * mean
    scale3 = (gamma.astype(jnp.float32) * lax.rsqrt(var + eps)).reshape(C, 1, 1)
    shift3 = (beta.astype(jnp.float32)
              - mean * scale3[:, 0, 0]).reshape(C, 1, 1)

    bn = 4
    while N % bn:
        bn -= 1
    out = pl.pallas_call(
        _bn_residual_kernel,
        out_shape=jax.ShapeDtypeStruct((N, C, H, W), jnp.float32),
        grid=(N // bn,),
        in_specs=[
            pl.BlockSpec((bn, C, H, W), lambda i: (i, 0, 0, 0)),
            pl.BlockSpec((bn, C, HW), lambda i: (i, 0, 0)),
            pl.BlockSpec((C, 1, 1), lambda i: (0, 0, 0)),
            pl.BlockSpec((C, 1, 1), lambda i: (0, 0, 0)),
        ],
        out_specs=pl.BlockSpec((bn, C, H, W), lambda i: (i, 0, 0, 0)),
        compiler_params=pltpu.CompilerParams(
            dimension_semantics=("parallel",)),
    )(xf, x3, scale3, shift3)
    return out


# speedup vs baseline: 2.3925x; 2.3925x over previous
import functools

import jax
import jax.numpy as jnp
from jax import lax
from jax.experimental import pallas as pl
from jax.experimental.pallas import tpu as pltpu


def _shift_lanes(a, s):
    if s > 0:
        return jnp.pad(a[:, s:], ((0, 0), (0, s)))
    if s < 0:
        return jnp.pad(a[:, :s], ((0, 0), (-s, 0)))
    return a


def _fused_conv_stats_kernel(x_ref, w1_ref, b1_ref, w3_ref, b3_ref,
                             x3_ref, st_ref, *, mid, height, width):
    hw = height * width
    xf = x_ref[0]
    x0 = jnp.dot(w1_ref[...], xf.astype(jnp.bfloat16),
                 preferred_element_type=jnp.float32) + b1_ref[...]
    x1 = x0[:mid].astype(jnp.bfloat16)

    pos = lax.broadcasted_iota(jnp.int32, (1, hw), 1)
    hidx = pos // width
    widx = pos - hidx * width
    taps = []
    for dh in (-1, 0, 1):
        for dw in (-1, 0, 1):
            shifted = _shift_lanes(x1, dh * width + dw)
            ok_h = jnp.logical_and(hidx + dh >= 0, hidx + dh < height)
            ok_w = jnp.logical_and(widx + dw >= 0, widx + dw < width)
            mask = jnp.logical_and(ok_h, ok_w)
            taps.append(jnp.where(mask, shifted,
                                  jnp.bfloat16(0.0)).astype(jnp.bfloat16))
    patch = jnp.concatenate(taps, axis=0)
    y = jnp.dot(w3_ref[...], patch,
                preferred_element_type=jnp.float32) + b3_ref[...]

    x3 = jnp.concatenate([y, x0[mid:]], axis=0)
    x3_ref[0] = x3
    st_ref[0] = jnp.concatenate(
        [jnp.sum(x3, axis=1, keepdims=True),
         jnp.sum(x3 * x3, axis=1, keepdims=True)], axis=1)


def _bn_residual_kernel(x_ref, x3_ref, scale_ref, shift_ref, o_ref):
    o_ref[...] = x_ref[...] + x3_ref[...] * scale_ref[...] + shift_ref[...]


def kernel(x, w1_t, b1, w3_t, b3, gamma, beta, eps=1e-5):
    N, C, H, W = x.shape
    mid = C // 2
    HW = H * W
    M = N * HW

    xf = x.reshape(N, C, HW).astype(jnp.float32)
    w1 = w1_t[:, :, 0, 0].astype(jnp.bfloat16)
    b1c = b1.reshape(C, 1).astype(jnp.float32)
    w3 = jnp.transpose(w3_t, (0, 2, 3, 1)).reshape(mid, 9 * mid)
    w3 = w3.astype(jnp.bfloat16)
    b3c = b3.reshape(mid, 1).astype(jnp.float32)

    kern = functools.partial(_fused_conv_stats_kernel,
                             mid=mid, height=H, width=W)
    x3, stats = pl.pallas_call(
        kern,
        out_shape=(jax.ShapeDtypeStruct((N, C, HW), jnp.float32),
                   jax.ShapeDtypeStruct((N, C, 2), jnp.float32)),
        grid=(N,),
        in_specs=[
            pl.BlockSpec((1, C, HW), lambda n: (n, 0, 0)),
            pl.BlockSpec((C, C), lambda n: (0, 0)),
            pl.BlockSpec((C, 1), lambda n: (0, 0)),
            pl.BlockSpec((mid, 9 * mid), lambda n: (0, 0)),
            pl.BlockSpec((mid, 1), lambda n: (0, 0)),
        ],
        out_specs=(
            pl.BlockSpec((1, C, HW), lambda n: (n, 0, 0)),
            pl.BlockSpec((1, C, 2), lambda n: (n, 0, 0)),
        ),
        compiler_params=pltpu.CompilerParams(
            dimension_semantics=("parallel",)),
    )(xf, w1, b1c, w3, b3c)

    tot = jnp.sum(stats, axis=0)
    mean = tot[:, 0] / M
    var = tot[:, 1] / M - mean * mean
    scale = (gamma.astype(jnp.float32) * lax.rsqrt(var + eps)).reshape(C, 1)
    shift = (beta.astype(jnp.float32) - mean * scale[:, 0]).reshape(C, 1)

    bn = 4
    while N % bn:
        bn -= 1
    out = pl.pallas_call(
        _bn_residual_kernel,
        out_shape=jax.ShapeDtypeStruct((N, C, HW), jnp.float32),
        grid=(N // bn,),
        in_specs=[
            pl.BlockSpec((bn, C, HW), lambda i: (i, 0, 0)),
            pl.BlockSpec((bn, C, HW), lambda i: (i, 0, 0)),
            pl.BlockSpec((C, 1), lambda i: (0, 0)),
            pl.BlockSpec((C, 1), lambda i: (0, 0)),
        ],
        out_specs=pl.BlockSpec((bn, C, HW), lambda i: (i, 0, 0)),
        compiler_params=pltpu.CompilerParams(
            dimension_semantics=("parallel",)),
    )(xf, x3, scale, shift)
    return out.reshape(N, C, H, W)


# speedup vs baseline: 2.6984x; 1.1278x over previous
import functools

import jax
import jax.numpy as jnp
from jax import lax
from jax.experimental import pallas as pl
from jax.experimental.pallas import tpu as pltpu


def _shift_lanes(a, s):
    if s > 0:
        return jnp.pad(a[:, s:], ((0, 0), (0, s)))
    if s < 0:
        return jnp.pad(a[:, :s], ((0, 0), (-s, 0)))
    return a


def _fused_conv_stats_kernel(x_ref, w1_ref, b1_ref, w3_ref, b3_ref, ones_ref,
                             x3_ref, st_ref, *, mid, height, width, nb):
    hw = height * width
    C = 2 * mid
    xb = jnp.concatenate([x_ref[i] for i in range(nb)], axis=1)
    x0 = jnp.dot(w1_ref[...], xb,
                 preferred_element_type=jnp.float32) + b1_ref[...]
    x1 = x0[:mid].astype(jnp.bfloat16)

    pos = lax.broadcasted_iota(jnp.int32, (1, nb * hw), 1)
    rem = pos % hw
    hidx = rem // width
    widx = rem - hidx * width
    cols = []
    for dw in (-1, 0, 1):
        c = _shift_lanes(x1, dw)
        if dw != 0:
            ok_w = jnp.logical_and(widx + dw >= 0, widx + dw < width)
            c = jnp.where(ok_w, c, jnp.bfloat16(0.0))
        cols.append(c)
    taps = []
    for dh in (-1, 0, 1):
        ok_h = jnp.logical_and(hidx + dh >= 0, hidx + dh < height)
        for c in cols:
            t = _shift_lanes(c, dh * width)
            taps.append(jnp.where(ok_h, t, jnp.bfloat16(0.0)))
    patch = jnp.concatenate(taps, axis=0)
    y = jnp.dot(w3_ref[...], patch,
                preferred_element_type=jnp.float32) + b3_ref[...]

    x3 = jnp.concatenate([y, x0[mid:]], axis=0)
    x3_ref[0] = x3.astype(jnp.bfloat16)

    both = jnp.concatenate([x3, x3 * x3], axis=0)
    sums = jnp.dot(both, ones_ref[...],
                   preferred_element_type=jnp.float32)
    st_ref[0] = jnp.concatenate([sums[:C], sums[C:]], axis=1)


def _bn_residual_kernel(x_ref, x3_ref, scale_ref, shift_ref, o_ref, *, nb):
    hw = o_ref.shape[2]
    x = jnp.concatenate([x_ref[i] for i in range(nb)], axis=1)
    res = (x.astype(jnp.float32)
           + x3_ref[0].astype(jnp.float32) * scale_ref[...] + shift_ref[...])
    for i in range(nb):
        o_ref[i] = res[:, i * hw:(i + 1) * hw]


def kernel(x, w1_t, b1, w3_t, b3, gamma, beta, eps=1e-5):
    N, C, H, W = x.shape
    mid = C // 2
    HW = H * W
    M = N * HW

    xb = x.reshape(N, C, HW).astype(jnp.bfloat16)
    w1 = w1_t[:, :, 0, 0].astype(jnp.bfloat16)
    b1c = b1.reshape(C, 1).astype(jnp.float32)
    w3 = jnp.transpose(w3_t, (0, 2, 3, 1)).reshape(mid, 9 * mid)
    w3 = w3.astype(jnp.bfloat16)
    b3c = b3.reshape(mid, 1).astype(jnp.float32)

    nb = 4
    while N % nb:
        nb -= 1
    ones = jnp.ones((nb * HW, 1), jnp.float32)

    kern = functools.partial(_fused_conv_stats_kernel,
                             mid=mid, height=H, width=W, nb=nb)
    x3, stats = pl.pallas_call(
        kern,
        out_shape=(jax.ShapeDtypeStruct((N // nb, C, nb * HW), jnp.bfloat16),
                   jax.ShapeDtypeStruct((N // nb, C, 2), jnp.float32)),
        grid=(N // nb,),
        in_specs=[
            pl.BlockSpec((nb, C, HW), lambda n: (n, 0, 0)),
            pl.BlockSpec((C, C), lambda n: (0, 0)),
            pl.BlockSpec((C, 1), lambda n: (0, 0)),
            pl.BlockSpec((mid, 9 * mid), lambda n: (0, 0)),
            pl.BlockSpec((mid, 1), lambda n: (0, 0)),
            pl.BlockSpec((nb * HW, 1), lambda n: (0, 0)),
        ],
        out_specs=(
            pl.BlockSpec((1, C, nb * HW), lambda n: (n, 0, 0)),
            pl.BlockSpec((1, C, 2), lambda n: (n, 0, 0)),
        ),
        compiler_params=pltpu.CompilerParams(
            dimension_semantics=("parallel",)),
    )(xb, w1, b1c, w3, b3c, ones)

    tot = jnp.sum(stats, axis=0)
    mean = tot[:, 0] / M
    var = tot[:, 1] / M - mean * mean
    scale = (gamma.astype(jnp.float32) * lax.rsqrt(var + eps)).reshape(C, 1)
    shift = (beta.astype(jnp.float32) - mean * scale[:, 0]).reshape(C, 1)

    out = pl.pallas_call(
        functools.partial(_bn_residual_kernel, nb=nb),
        out_shape=jax.ShapeDtypeStruct((N, C, HW), jnp.float32),
        grid=(N // nb,),
        in_specs=[
            pl.BlockSpec((nb, C, HW), lambda i: (i, 0, 0)),
            pl.BlockSpec((1, C, nb * HW), lambda i: (i, 0, 0)),
            pl.BlockSpec((C, 1), lambda i: (0, 0)),
            pl.BlockSpec((C, 1), lambda i: (0, 0)),
        ],
        out_specs=pl.BlockSpec((nb, C, HW), lambda i: (i, 0, 0)),
        compiler_params=pltpu.CompilerParams(
            dimension_semantics=("parallel",)),
    )(xb, x3, scale, shift)
    return out.reshape(N, C, H, W)


# speedup vs baseline: 4.0591x; 1.5043x over previous
import functools

import jax
import jax.numpy as jnp
from jax import lax
from jax.experimental import pallas as pl
from jax.experimental.pallas import tpu as pltpu


def _shift_rows(a, s):
    if s > 0:
        return jnp.pad(a[s:], ((0, s), (0, 0)))
    if s < 0:
        return jnp.pad(a[:s], ((-s, 0), (0, 0)))
    return a


def _fused_conv_stats_kernel(x_ref, w1_ref, b1_ref, w3_ref, b3_ref,
                             x3_ref, st_ref, *, mid, height, width, rm):
    hw = height * width
    xb = x_ref[...].astype(jnp.bfloat16)
    x0 = jnp.dot(xb, w1_ref[...],
                 preferred_element_type=jnp.float32) + b1_ref[...]
    x1 = x0[:, :mid].astype(jnp.bfloat16)

    pos = lax.broadcasted_iota(jnp.int32, (rm, 1), 0)
    rem = pos % hw
    hidx = rem // width
    widx = rem - hidx * width
    cols = []
    for dw in (-1, 0, 1):
        c = _shift_rows(x1, dw)
        if dw != 0:
            ok_w = jnp.logical_and(widx + dw >= 0, widx + dw < width)
            c = jnp.where(ok_w, c, jnp.bfloat16(0.0))
        cols.append(c)
    taps = []
    for dh in (-1, 0, 1):
        ok_h = jnp.logical_and(hidx + dh >= 0, hidx + dh < height)
        for c in cols:
            t = _shift_rows(c, dh * width)
            taps.append(jnp.where(ok_h, t, jnp.bfloat16(0.0)))
    patch = jnp.concatenate(taps, axis=1)
    y = jnp.dot(patch, w3_ref[...],
                preferred_element_type=jnp.float32) + b3_ref[...]

    x3 = jnp.concatenate([y, x0[:, mid:]], axis=1)
    x3_ref[...] = x3.astype(jnp.bfloat16)

    st_ref[0] = jnp.concatenate(
        [jnp.sum(x3, axis=0, keepdims=True),
         jnp.sum(x3 * x3, axis=0, keepdims=True)], axis=0)


def _bn_residual_kernel(x_ref, x3_ref, scale_ref, shift_ref, o_ref):
    x3 = x3_ref[...].astype(jnp.float32)
    o_ref[...] = x_ref[...] + x3 * scale_ref[...] + shift_ref[...]


def kernel(x, w1_t, b1, w3_t, b3, gamma, beta, eps=1e-5):
    N, C, H, W = x.shape
    mid = C // 2
    HW = H * W
    M = N * HW

    x2d = jnp.transpose(x, (0, 2, 3, 1)).reshape(M, C).astype(jnp.float32)
    w1 = jnp.transpose(w1_t[:, :, 0, 0]).astype(jnp.bfloat16)
    b1r = b1.reshape(1, C).astype(jnp.float32)
    w3 = jnp.transpose(w3_t, (2, 3, 1, 0)).reshape(9 * mid, mid)
    w3 = w3.astype(jnp.bfloat16)
    b3r = b3.reshape(1, mid).astype(jnp.float32)

    nb = 4
    while N % nb:
        nb -= 1
    RM = nb * HW
    G = M // RM

    kern = functools.partial(_fused_conv_stats_kernel,
                             mid=mid, height=H, width=W, rm=RM)
    x3, stats = pl.pallas_call(
        kern,
        out_shape=(jax.ShapeDtypeStruct((M, C), jnp.bfloat16),
                   jax.ShapeDtypeStruct((G, 2, C), jnp.float32)),
        grid=(G,),
        in_specs=[
            pl.BlockSpec((RM, C), lambda g: (g, 0)),
            pl.BlockSpec((C, C), lambda g: (0, 0)),
            pl.BlockSpec((1, C), lambda g: (0, 0)),
            pl.BlockSpec((9 * mid, mid), lambda g: (0, 0)),
            pl.BlockSpec((1, mid), lambda g: (0, 0)),
        ],
        out_specs=(
            pl.BlockSpec((RM, C), lambda g: (g, 0)),
            pl.BlockSpec((1, 2, C), lambda g: (g, 0, 0)),
        ),
        compiler_params=pltpu.CompilerParams(
            dimension_semantics=("parallel",)),
    )(x2d, w1, b1r, w3, b3r)

    tot = jnp.sum(stats, axis=0)
    mean = tot[0] / M
    var = tot[1] / M - mean * mean
    scale = (gamma.astype(jnp.float32) * lax.rsqrt(var + eps)).reshape(1, C)
    shift = (beta.astype(jnp.float32) - mean * scale[0]).reshape(1, C)

    BM = RM
    out = pl.pallas_call(
        _bn_residual_kernel,
        out_shape=jax.ShapeDtypeStruct((M, C), jnp.float32),
        grid=(M // BM,),
        in_specs=[
            pl.BlockSpec((BM, C), lambda i: (i, 0)),
            pl.BlockSpec((BM, C), lambda i: (i, 0)),
            pl.BlockSpec((1, C), lambda i: (0, 0)),
            pl.BlockSpec((1, C), lambda i: (0, 0)),
        ],
        out_specs=pl.BlockSpec((BM, C), lambda i: (i, 0)),
        compiler_params=pltpu.CompilerParams(
            dimension_semantics=("parallel",)),
    )(x2d, x3, scale, shift)
    return jnp.transpose(out.reshape(N, H, W, C), (0, 3, 1, 2))


# speedup vs baseline: 5.7985x; 1.4285x over previous
import functools

import jax
import jax.numpy as jnp
from jax import lax
from jax.experimental import pallas as pl
from jax.experimental.pallas import tpu as pltpu


def _shift_rows(a, s):
    if s > 0:
        return jnp.pad(a[s:], ((0, s), (0, 0)))
    if s < 0:
        return jnp.pad(a[:s], ((-s, 0), (0, 0)))
    return a


def _fused_conv_stats_kernel(x_ref, w1_ref, b1_ref, w3_ref, b3_ref,
                             x3_ref, st_ref, xb_ref, *, mid, height, width,
                             rm):
    hw = height * width
    xb = x_ref[...].astype(jnp.bfloat16)
    xb_ref[...] = xb
    x0 = jnp.dot(xb, w1_ref[...],
                 preferred_element_type=jnp.float32) + b1_ref[...]
    x1 = x0[:, :mid].astype(jnp.bfloat16)

    pos = lax.broadcasted_iota(jnp.int32, (rm, 1), 0)
    rem = pos % hw
    hidx = rem // width
    widx = rem - hidx * width

    cw = [jnp.where(widx >= 1, _shift_rows(x1, -1), jnp.bfloat16(0.0)),
          x1,
          jnp.where(widx < width - 1, _shift_rows(x1, 1), jnp.bfloat16(0.0))]
    cols = jnp.concatenate(cw, axis=1)
    z = jnp.dot(cols, w3_ref[...],
                preferred_element_type=jnp.float32)
    y = (jnp.where(hidx >= 1, _shift_rows(z[:, :mid], -width), 0.0)
         + z[:, mid:2 * mid]
         + jnp.where(hidx < height - 1,
                     _shift_rows(z[:, 2 * mid:], width), 0.0)
         + b3_ref[...])

    x3 = jnp.concatenate([y, x0[:, mid:]], axis=1)
    x3_ref[...] = x3.astype(jnp.bfloat16)

    st_ref[0] = jnp.concatenate(
        [jnp.sum(x3, axis=0, keepdims=True),
         jnp.sum(x3 * x3, axis=0, keepdims=True)], axis=0)


def _bn_residual_kernel(x_ref, x3_ref, scale_ref, shift_ref, o_ref):
    x = x_ref[...].astype(jnp.float32)
    x3 = x3_ref[...].astype(jnp.float32)
    o_ref[...] = x + x3 * scale_ref[...] + shift_ref[...]


def kernel(x, w1_t, b1, w3_t, b3, gamma, beta, eps=1e-5):
    N, C, H, W = x.shape
    mid = C // 2
    HW = H * W
    M = N * HW

    x2d = jnp.transpose(x, (0, 2, 3, 1)).reshape(M, C).astype(jnp.float32)
    w1 = jnp.transpose(w1_t[:, :, 0, 0]).astype(jnp.bfloat16)
    b1r = b1.reshape(1, C).astype(jnp.float32)
    w3b = jnp.transpose(w3_t, (2, 3, 1, 0)).reshape(3, 3 * mid, mid)
    w3 = jnp.concatenate([w3b[0], w3b[1], w3b[2]], axis=1)
    w3 = w3.astype(jnp.bfloat16)
    b3r = b3.reshape(1, mid).astype(jnp.float32)

    nb = 4
    while N % nb:
        nb -= 1
    RM = nb * HW
    G = M // RM

    kern = functools.partial(_fused_conv_stats_kernel,
                             mid=mid, height=H, width=W, rm=RM)
    x3, stats, xb = pl.pallas_call(
        kern,
        out_shape=(jax.ShapeDtypeStruct((M, C), jnp.bfloat16),
                   jax.ShapeDtypeStruct((G, 2, C), jnp.float32),
                   jax.ShapeDtypeStruct((M, C), jnp.bfloat16)),
        grid=(G,),
        in_specs=[
            pl.BlockSpec((RM, C), lambda g: (g, 0)),
            pl.BlockSpec((C, C), lambda g: (0, 0)),
            pl.BlockSpec((1, C), lambda g: (0, 0)),
            pl.BlockSpec((3 * mid, 3 * mid), lambda g: (0, 0)),
            pl.BlockSpec((1, mid), lambda g: (0, 0)),
        ],
        out_specs=(
            pl.BlockSpec((RM, C), lambda g: (g, 0)),
            pl.BlockSpec((1, 2, C), lambda g: (g, 0, 0)),
            pl.BlockSpec((RM, C), lambda g: (g, 0)),
        ),
        compiler_params=pltpu.CompilerParams(
            dimension_semantics=("parallel",)),
    )(x2d, w1, b1r, w3, b3r)

    tot = jnp.sum(stats, axis=0)
    mean = tot[0] / M
    var = tot[1] / M - mean * mean
    scale = (gamma.astype(jnp.float32) * lax.rsqrt(var + eps)).reshape(1, C)
    shift = (beta.astype(jnp.float32) - mean * scale[0]).reshape(1, C)

    BM = RM
    out = pl.pallas_call(
        _bn_residual_kernel,
        out_shape=jax.ShapeDtypeStruct((M, C), jnp.float32),
        grid=(M // BM,),
        in_specs=[
            pl.BlockSpec((BM, C), lambda i: (i, 0)),
            pl.BlockSpec((BM, C), lambda i: (i, 0)),
            pl.BlockSpec((1, C), lambda i: (0, 0)),
            pl.BlockSpec((1, C), lambda i: (0, 0)),
        ],
        out_specs=pl.BlockSpec((BM, C), lambda i: (i, 0)),
        compiler_params=pltpu.CompilerParams(
            dimension_semantics=("parallel",)),
    )(xb, x3, scale, shift)
    return jnp.transpose(out.reshape(N, H, W, C), (0, 3, 1, 2))


# speedup vs baseline: 6.5972x; 1.1377x over previous
import functools

import jax
import jax.numpy as jnp
from jax import lax
from jax.experimental import pallas as pl
from jax.experimental.pallas import tpu as pltpu


def _shift_rows(a, s):
    if s > 0:
        return jnp.pad(a[s:], ((0, s), (0, 0)))
    if s < 0:
        return jnp.pad(a[:s], ((-s, 0), (0, 0)))
    return a


def _fused_kernel(x_ref, w1_ref, b1_ref, w3_ref, b3_ref, g_ref, be_ref,
                  o_ref, x3s, xbs, st, sc, *, mid, height, width, rm,
                  nsteps, inv_m, eps):
    g = pl.program_id(0)
    hw = height * width

    @pl.when(g < nsteps)
    def _compute():
        xb = x_ref[...].astype(jnp.bfloat16)
        xbs[pl.ds(g * rm, rm), :] = xb
        x0 = jnp.dot(xb, w1_ref[...],
                     preferred_element_type=jnp.float32) + b1_ref[...]
        x1 = x0[:, :mid].astype(jnp.bfloat16)

        pos = lax.broadcasted_iota(jnp.int32, (rm, 1), 0)
        rem = pos % hw
        hidx = rem // width
        widx = rem - hidx * width

        cw = [jnp.where(widx >= 1, _shift_rows(x1, -1), jnp.bfloat16(0.0)),
              x1,
              jnp.where(widx < width - 1, _shift_rows(x1, 1),
                        jnp.bfloat16(0.0))]
        cols = jnp.concatenate(cw, axis=1)
        z = jnp.dot(cols, w3_ref[...],
                    preferred_element_type=jnp.float32)
        y = (jnp.where(hidx >= 1, _shift_rows(z[:, :mid], -width), 0.0)
             + z[:, mid:2 * mid]
             + jnp.where(hidx < height - 1,
                         _shift_rows(z[:, 2 * mid:], width), 0.0)
             + b3_ref[...])

        x3 = jnp.concatenate([y, x0[:, mid:]], axis=1)
        x3s[pl.ds(g * rm, rm), :] = x3.astype(jnp.bfloat16)

        tile_stats = jnp.concatenate(
            [jnp.sum(x3, axis=0, keepdims=True),
             jnp.sum(x3 * x3, axis=0, keepdims=True)], axis=0)

        @pl.when(g == 0)
        def _():
            st[...] = jnp.zeros_like(st)

        st[...] += tile_stats

    @pl.when(g == nsteps)
    def _fold_stats():
        mean = st[0:1, :] * inv_m
        var = st[1:2, :] * inv_m - mean * mean
        scale = g_ref[...] * lax.rsqrt(var + eps)
        sc[0:1, :] = scale
        sc[1:2, :] = be_ref[...] - mean * scale

    @pl.when(g >= nsteps)
    def _apply():
        i = g - nsteps
        xb = xbs[pl.ds(i * rm, rm), :].astype(jnp.float32)
        x3 = x3s[pl.ds(i * rm, rm), :].astype(jnp.float32)
        o_ref[...] = xb + x3 * sc[0:1, :] + sc[1:2, :]


def kernel(x, w1_t, b1, w3_t, b3, gamma, beta, eps=1e-5):
    N, C, H, W = x.shape
    mid = C // 2
    HW = H * W
    M = N * HW

    x2d = jnp.transpose(x, (0, 2, 3, 1)).reshape(M, C).astype(jnp.float32)
    w1 = jnp.transpose(w1_t[:, :, 0, 0]).astype(jnp.bfloat16)
    b1r = b1.reshape(1, C).astype(jnp.float32)
    w3b = jnp.transpose(w3_t, (2, 3, 1, 0)).reshape(3, 3 * mid, mid)
    w3 = jnp.concatenate([w3b[0], w3b[1], w3b[2]], axis=1)
    w3 = w3.astype(jnp.bfloat16)
    b3r = b3.reshape(1, mid).astype(jnp.float32)
    gr = gamma.reshape(1, C).astype(jnp.float32)
    ber = beta.reshape(1, C).astype(jnp.float32)

    nb = 2
    while N % nb:
        nb -= 1
    RM = nb * HW
    G = M // RM

    kern = functools.partial(_fused_kernel, mid=mid, height=H, width=W,
                             rm=RM, nsteps=G, inv_m=1.0 / M, eps=eps)
    out = pl.pallas_call(
        kern,
        out_shape=jax.ShapeDtypeStruct((M, C), jnp.float32),
        grid=(2 * G,),
        in_specs=[
            pl.BlockSpec((RM, C), lambda g: (jnp.where(g < G, g, 0), 0)),
            pl.BlockSpec((C, C), lambda g: (0, 0)),
            pl.BlockSpec((1, C), lambda g: (0, 0)),
            pl.BlockSpec((3 * mid, 3 * mid), lambda g: (0, 0)),
            pl.BlockSpec((1, mid), lambda g: (0, 0)),
            pl.BlockSpec((1, C), lambda g: (0, 0)),
            pl.BlockSpec((1, C), lambda g: (0, 0)),
        ],
        out_specs=pl.BlockSpec(
            (RM, C), lambda g: (jnp.where(g < G, 0, g - G), 0)),
        scratch_shapes=[
            pltpu.VMEM((M, C), jnp.bfloat16),
            pltpu.VMEM((M, C), jnp.bfloat16),
            pltpu.VMEM((2, C), jnp.float32),
            pltpu.VMEM((2, C), jnp.float32),
        ],
        compiler_params=pltpu.CompilerParams(
            dimension_semantics=("arbitrary",)),
    )(x2d, w1, b1r, w3, b3r, gr, ber)
    return jnp.transpose(out.reshape(N, H, W, C), (0, 3, 1, 2))


# speedup vs baseline: 7.3869x; 1.1197x over previous
import functools

import jax
import jax.numpy as jnp
from jax import lax
from jax.experimental import pallas as pl
from jax.experimental.pallas import tpu as pltpu


def _shift_rows(a, s):
    if s > 0:
        return jnp.pad(a[s:], ((0, s), (0, 0)))
    if s < 0:
        return jnp.pad(a[:s], ((-s, 0), (0, 0)))
    return a


def _fused_kernel(x_ref, w1_ref, b1_ref, w3_ref, b3_ref, g_ref, be_ref,
                  o_ref, x3s, xbs, st, sc, *, mid, height, width, rm,
                  nsteps, inv_m, eps):
    g = pl.program_id(0)
    hw = height * width

    @pl.when(g < nsteps)
    def _compute():
        xb = x_ref[...].astype(jnp.bfloat16)
        xbs[pl.ds(g * rm, rm), :] = xb
        x0 = jnp.dot(xb, w1_ref[...],
                     preferred_element_type=jnp.float32) + b1_ref[...]
        x1 = x0[:, :mid].astype(jnp.bfloat16)

        pos = lax.broadcasted_iota(jnp.int32, (rm, 1), 0)
        rem = pos % hw
        hidx = rem // width
        widx = rem - hidx * width

        cw = [jnp.where(widx >= 1, _shift_rows(x1, -1), jnp.bfloat16(0.0)),
              x1,
              jnp.where(widx < width - 1, _shift_rows(x1, 1),
                        jnp.bfloat16(0.0))]
        cols = jnp.concatenate(cw, axis=1)
        z = jnp.dot(cols, w3_ref[...],
                    preferred_element_type=jnp.float32)
        y = (jnp.where(hidx >= 1, _shift_rows(z[:, :mid], -width), 0.0)
             + z[:, mid:2 * mid]
             + jnp.where(hidx < height - 1,
                         _shift_rows(z[:, 2 * mid:], width), 0.0)
             + b3_ref[...])

        x3 = jnp.concatenate([y, x0[:, mid:]], axis=1)
        x3s[pl.ds(g * rm, rm), :] = x3.astype(jnp.bfloat16)

        tile_stats = jnp.concatenate(
            [jnp.sum(x3, axis=0, keepdims=True),
             jnp.sum(x3 * x3, axis=0, keepdims=True)], axis=0)

        @pl.when(g == 0)
        def _():
            st[...] = jnp.zeros_like(st)

        st[...] += tile_stats

    @pl.when(g == nsteps)
    def _fold_stats():
        mean = st[0:1, :] * inv_m
        var = st[1:2, :] * inv_m - mean * mean
        scale = g_ref[...] * lax.rsqrt(var + eps)
        sc[0:1, :] = scale
        sc[1:2, :] = be_ref[...] - mean * scale

    @pl.when(g >= nsteps)
    def _apply():
        i = g - nsteps
        xb = xbs[pl.ds(i * rm, rm), :].astype(jnp.float32)
        x3 = x3s[pl.ds(i * rm, rm), :].astype(jnp.float32)
        o_ref[...] = xb + x3 * sc[0:1, :] + sc[1:2, :]


def kernel(x, w1_t, b1, w3_t, b3, gamma, beta, eps=1e-5):
    N, C, H, W = x.shape
    mid = C // 2
    HW = H * W
    M = N * HW

    x2d = jnp.transpose(x, (0, 2, 3, 1)).reshape(M, C).astype(jnp.float32)
    w1 = jnp.transpose(w1_t[:, :, 0, 0]).astype(jnp.bfloat16)
    b1r = b1.reshape(1, C).astype(jnp.float32)
    w3b = jnp.transpose(w3_t, (2, 3, 1, 0)).reshape(3, 3 * mid, mid)
    w3 = jnp.concatenate([w3b[0], w3b[1], w3b[2]], axis=1)
    w3 = w3.astype(jnp.bfloat16)
    b3r = b3.reshape(1, mid).astype(jnp.float32)
    gr = gamma.reshape(1, C).astype(jnp.float32)
    ber = beta.reshape(1, C).astype(jnp.float32)

    nb = 4
    while N % nb:
        nb -= 1
    RM = nb * HW
    G = M // RM

    kern = functools.partial(_fused_kernel, mid=mid, height=H, width=W,
                             rm=RM, nsteps=G, inv_m=1.0 / M, eps=eps)
    out = pl.pallas_call(
        kern,
        out_shape=jax.ShapeDtypeStruct((M, C), jnp.float32),
        grid=(2 * G,),
        in_specs=[
            pl.BlockSpec((RM, C), lambda g: (jnp.where(g < G, g, 0), 0)),
            pl.BlockSpec((C, C), lambda g: (0, 0)),
            pl.BlockSpec((1, C), lambda g: (0, 0)),
            pl.BlockSpec((3 * mid, 3 * mid), lambda g: (0, 0)),
            pl.BlockSpec((1, mid), lambda g: (0, 0)),
            pl.BlockSpec((1, C), lambda g: (0, 0)),
            pl.BlockSpec((1, C), lambda g: (0, 0)),
        ],
        out_specs=pl.BlockSpec(
            (RM, C), lambda g: (jnp.where(g < G, 0, g - G), 0)),
        scratch_shapes=[
            pltpu.VMEM((M, C), jnp.bfloat16),
            pltpu.VMEM((M, C), jnp.bfloat16),
            pltpu.VMEM((2, C), jnp.float32),
            pltpu.VMEM((2, C), jnp.float32),
        ],
        compiler_params=pltpu.CompilerParams(
            dimension_semantics=("arbitrary",)),
    )(x2d, w1, b1r, w3, b3r, gr, ber)
    return jnp.transpose(out.reshape(N, H, W, C), (0, 3, 1, 2))


# speedup vs baseline: 8.0122x; 1.0847x over previous
import functools

import jax
import jax.numpy as jnp
from jax import lax
from jax.experimental import pallas as pl
from jax.experimental.pallas import tpu as pltpu


def _shift_rows(a, s):
    if s > 0:
        return jnp.pad(a[s:], ((0, s), (0, 0)))
    if s < 0:
        return jnp.pad(a[:s], ((-s, 0), (0, 0)))
    return a


def _fused_kernel(x_ref, w1_ref, b1_ref, w3_ref, b3_ref, g_ref, be_ref,
                  o_ref, x3s, xbs, st, sc, *, mid, height, width, rm,
                  nsteps, inv_m, eps):
    g = pl.program_id(0)
    hw = height * width

    @pl.when(g < nsteps)
    def _compute():
        xb = x_ref[...].astype(jnp.bfloat16)
        xbs[pl.ds(g * rm, rm), :] = xb
        x0 = jnp.dot(xb, w1_ref[...],
                     preferred_element_type=jnp.float32) + b1_ref[...]
        x1 = x0[:, :mid].astype(jnp.bfloat16)

        pos = lax.broadcasted_iota(jnp.int32, (rm, 1), 0)
        rem = pos % hw
        hidx = rem // width
        widx = rem - hidx * width

        cw = [jnp.where(widx >= 1, _shift_rows(x1, -1), jnp.bfloat16(0.0)),
              x1,
              jnp.where(widx < width - 1, _shift_rows(x1, 1),
                        jnp.bfloat16(0.0))]
        cols = jnp.concatenate(cw, axis=1)
        z = jnp.dot(cols, w3_ref[...],
                    preferred_element_type=jnp.float32)
        y = (jnp.where(hidx >= 1, _shift_rows(z[:, :mid], -width), 0.0)
             + z[:, mid:2 * mid]
             + jnp.where(hidx < height - 1,
                         _shift_rows(z[:, 2 * mid:], width), 0.0)
             + b3_ref[...])

        x3 = jnp.concatenate([y, x0[:, mid:]], axis=1)
        x3s[pl.ds(g * rm, rm), :] = x3.astype(jnp.bfloat16)

        tile_stats = jnp.concatenate(
            [jnp.sum(x3, axis=0, keepdims=True),
             jnp.sum(x3 * x3, axis=0, keepdims=True)], axis=0)

        @pl.when(g == 0)
        def _():
            st[...] = jnp.zeros_like(st)

        st[...] += tile_stats

    @pl.when(g == nsteps)
    def _fold_stats():
        mean = st[0:1, :] * inv_m
        var = st[1:2, :] * inv_m - mean * mean
        scale = g_ref[...] * lax.rsqrt(var + eps)
        sc[0:1, :] = scale
        sc[1:2, :] = be_ref[...] - mean * scale

    @pl.when(g >= nsteps)
    def _apply():
        i = g - nsteps
        xb = xbs[pl.ds(i * rm, rm), :].astype(jnp.float32)
        x3 = x3s[pl.ds(i * rm, rm), :].astype(jnp.float32)
        o_ref[...] = xb + x3 * sc[0:1, :] + sc[1:2, :]


def kernel(x, w1_t, b1, w3_t, b3, gamma, beta, eps=1e-5):
    N, C, H, W = x.shape
    mid = C // 2
    HW = H * W
    M = N * HW

    x2d = jnp.transpose(x, (0, 2, 3, 1)).reshape(M, C).astype(jnp.float32)
    w1 = jnp.transpose(w1_t[:, :, 0, 0]).astype(jnp.bfloat16)
    b1r = b1.reshape(1, C).astype(jnp.float32)
    w3b = jnp.transpose(w3_t, (2, 3, 1, 0)).reshape(3, 3 * mid, mid)
    w3 = jnp.concatenate([w3b[0], w3b[1], w3b[2]], axis=1)
    w3 = w3.astype(jnp.bfloat16)
    b3r = b3.reshape(1, mid).astype(jnp.float32)
    gr = gamma.reshape(1, C).astype(jnp.float32)
    ber = beta.reshape(1, C).astype(jnp.float32)

    nb = 8
    while N % nb:
        nb -= 1
    RM = nb * HW
    G = M // RM

    kern = functools.partial(_fused_kernel, mid=mid, height=H, width=W,
                             rm=RM, nsteps=G, inv_m=1.0 / M, eps=eps)
    out = pl.pallas_call(
        kern,
        out_shape=jax.ShapeDtypeStruct((M, C), jnp.float32),
        grid=(2 * G,),
        in_specs=[
            pl.BlockSpec((RM, C), lambda g: (jnp.where(g < G, g, 0), 0)),
            pl.BlockSpec((C, C), lambda g: (0, 0)),
            pl.BlockSpec((1, C), lambda g: (0, 0)),
            pl.BlockSpec((3 * mid, 3 * mid), lambda g: (0, 0)),
            pl.BlockSpec((1, mid), lambda g: (0, 0)),
            pl.BlockSpec((1, C), lambda g: (0, 0)),
            pl.BlockSpec((1, C), lambda g: (0, 0)),
        ],
        out_specs=pl.BlockSpec(
            (RM, C), lambda g: (jnp.where(g < G, 0, g - G), 0)),
        scratch_shapes=[
            pltpu.VMEM((M, C), jnp.bfloat16),
            pltpu.VMEM((M, C), jnp.bfloat16),
            pltpu.VMEM((2, C), jnp.float32),
            pltpu.VMEM((2, C), jnp.float32),
        ],
        compiler_params=pltpu.CompilerParams(
            dimension_semantics=("arbitrary",),
            vmem_limit_bytes=62 * 1024 * 1024),
    )(x2d, w1, b1r, w3, b3r, gr, ber)
    return jnp.transpose(out.reshape(N, H, W, C), (0, 3, 1, 2))
